# Initial kernel scaffold; baseline (speedup 1.0000x reference)
import jax
import jax.numpy as jnp
from jax.experimental import pallas as pl


def kernel(tokens, main_w, shared_w, size_w):
    raise NotImplementedError("write your pallas kernel here")


# speedup vs baseline: 1.4876x; 1.4876x over previous
import functools

import jax
import jax.numpy as jnp
from jax import lax
from jax.experimental import pallas as pl
from jax.experimental.pallas import tpu as pltpu
from jax.experimental.pallas import tpu_sc as plsc

EMBED_DIM = 128
MAX_N = 8
NUM_BUCKETS = 500000
HASH_BASE = 260
HASH_MOD = 1 << 23
MASK = HASH_MOD - 1

BSZ = 16
SEQ = 2048
NPOS = BSZ * SEQ
NTAB = 1 + (MAX_N - 2)
NW = 32
C = 64
NCHUNKS = NPOS // C
CH_PER_W = NCHUNKS // NW


def _idx_body(tok_ref, idx_ref):
    t = tok_ref[...]
    idx_ref[0] = t
    col = lax.broadcasted_iota(jnp.int32, (BSZ, SEQ), 1)
    s_acc = t
    for n in range(2, MAX_N + 1):
        d = n - 1
        p = pow(HASH_BASE, d, HASH_MOD)
        t_shift = jnp.concatenate(
            [jnp.zeros((BSZ, d), jnp.int32), t[:, : SEQ - d]], axis=1)
        s_acc = s_acc + ((t_shift * p) & MASK)
        if n >= 3:
            idx = (s_acc & MASK) % NUM_BUCKETS
            idx_ref[n - 2] = jnp.where(col < n - 1, 0, idx)


def _compute_indices(tokens):
    return pl.pallas_call(
        _idx_body,
        out_shape=jax.ShapeDtypeStruct((NTAB, BSZ, SEQ), jnp.int32),
    )(tokens)


def _sc_body(idx_hbm, main_hbm, shared_hbm, size_hbm, out_hbm,
             idx_v, b0, b1, b2, b3, b4, b5, b6, outb, size_v, gsem):
    bufs = (b0, b1, b2, b3, b4, b5, b6)
    wid = lax.axis_index("s") * 2 + lax.axis_index("c")

    pltpu.sync_copy(size_hbm, size_v)
    const = []
    for v in range(EMBED_DIM // 16):
        sl = pl.ds(v * 16, 16)
        cv = size_v[0, sl]
        for t in range(1, MAX_N - 2):
            cv = cv + size_v[t, sl]
        const.append(cv)

    for ch in range(CH_PER_W):
        chunk = wid * CH_PER_W + ch
        pltpu.sync_copy(idx_hbm.at[chunk], idx_v)
        cps = []
        for t in range(NTAB):
            table = main_hbm if t == 0 else shared_hbm
            cps.append(pltpu.async_copy(table.at[idx_v.at[t]], bufs[t], gsem))
        for cp in cps:
            cp.wait()

        def body(r, cc):
            for v in range(EMBED_DIM // 16):
                sl = pl.ds(v * 16, 16)
                acc = bufs[0][r, sl]
                for t in range(1, NTAB):
                    acc = acc + bufs[t][r, sl]
                outb[r, sl] = (acc + cc[v]) * (1.0 / 7.0)
            return cc

        lax.fori_loop(0, C, body, tuple(const))
        pltpu.sync_copy(outb, out_hbm.at[pl.ds(chunk * C, C)])


@functools.cache
def _sc_gather_sum():
    return pl.kernel(
        _sc_body,
        out_type=jax.ShapeDtypeStruct((NPOS, EMBED_DIM), jnp.float32),
        mesh=plsc.VectorSubcoreMesh(core_axis_name="c", subcore_axis_name="s"),
        scratch_types=(
            [pltpu.VMEM((NTAB, C), jnp.int32)]
            + [pltpu.VMEM((C, EMBED_DIM), jnp.float32) for _ in range(NTAB)]
            + [pltpu.VMEM((C, EMBED_DIM), jnp.float32),
               pltpu.VMEM((MAX_N - 2, EMBED_DIM), jnp.float32),
               pltpu.SemaphoreType.DMA]
        ),
    )


def kernel(tokens, main_w, shared_w, size_w):
    tokens = tokens.astype(jnp.int32)
    idx = _compute_indices(tokens)
    idx = idx.reshape(NTAB, NCHUNKS, C).transpose(1, 0, 2)
    out = _sc_gather_sum()(idx, main_w, shared_w, size_w)
    return out.reshape(BSZ, SEQ, EMBED_DIM)


# speedup vs baseline: 1.9139x; 1.2865x over previous
import functools

import jax
import jax.numpy as jnp
from jax import lax
from jax.experimental import pallas as pl
from jax.experimental.pallas import tpu as pltpu
from jax.experimental.pallas import tpu_sc as plsc

EMBED_DIM = 128
MAX_N = 8
NUM_BUCKETS = 500000
HASH_BASE = 260
HASH_MOD = 1 << 23
MASK = HASH_MOD - 1

BSZ = 16
SEQ = 2048
NPOS = BSZ * SEQ
NTAB = 1 + (MAX_N - 2)
NW = 32
C = 32
NCHUNKS = NPOS // C
CH_PER_W = NCHUNKS // NW


def _idx_body(tok_ref, idx_ref):
    t = tok_ref[...]
    idx_ref[0] = t
    col = lax.broadcasted_iota(jnp.int32, (BSZ, SEQ), 1)
    s_acc = t
    for n in range(2, MAX_N + 1):
        d = n - 1
        p = pow(HASH_BASE, d, HASH_MOD)
        t_shift = jnp.concatenate(
            [jnp.zeros((BSZ, d), jnp.int32), t[:, : SEQ - d]], axis=1)
        s_acc = s_acc + ((t_shift * p) & MASK)
        if n >= 3:
            idx = (s_acc & MASK) % NUM_BUCKETS
            idx_ref[n - 2] = jnp.where(col < n - 1, 0, idx)


def _compute_indices(tokens):
    return pl.pallas_call(
        _idx_body,
        out_shape=jax.ShapeDtypeStruct((NTAB, BSZ, SEQ), jnp.int32),
    )(tokens)


def _sc_body(idx_hbm, main_hbm, shared_hbm, size_hbm, out_hbm, *s):
    idxv = (s[0], s[1])
    bufs = (s[2:2 + NTAB], s[2 + NTAB:2 + 2 * NTAB])
    outb = (s[2 + 2 * NTAB], s[3 + 2 * NTAB])
    size_v = s[4 + 2 * NTAB]
    gsem, isem = s[5 + 2 * NTAB], s[6 + 2 * NTAB]
    osem = (s[7 + 2 * NTAB], s[8 + 2 * NTAB])
    wid = lax.axis_index("s") * 2 + lax.axis_index("c")

    pltpu.sync_copy(size_hbm, size_v)
    const = []
    for v in range(EMBED_DIM // 16):
        sl = pl.ds(v * 16, 16)
        cv = size_v[0, sl]
        for t in range(1, MAX_N - 2):
            cv = cv + size_v[t, sl]
        const.append(cv)

    def fire_gathers(par):
        cps = []
        for t in range(NTAB):
            table = main_hbm if t == 0 else shared_hbm
            cps.append(pltpu.async_copy(
                table.at[idxv[par].at[t]], bufs[par][t], gsem))
        return cps

    chunk0 = wid * CH_PER_W
    pltpu.sync_copy(idx_hbm.at[chunk0], idxv[0])
    g = {0: fire_gathers(0)}
    i_cp = pltpu.async_copy(idx_hbm.at[chunk0 + 1], idxv[1], isem)
    o_cp = {}

    for ch in range(CH_PER_W):
        par = ch % 2
        if ch + 1 < CH_PER_W:
            i_cp.wait()
            g[ch + 1] = fire_gathers(1 - par)
        for cp in g.pop(ch):
            cp.wait()
        if ch + 2 < CH_PER_W:
            i_cp = pltpu.async_copy(
                idx_hbm.at[chunk0 + ch + 2], idxv[par], isem)
        if ch - 2 in o_cp:
            o_cp.pop(ch - 2).wait()

        def body(r, cc, par=par):
            for v in range(EMBED_DIM // 16):
                sl = pl.ds(v * 16, 16)
                acc = bufs[par][0][r, sl]
                for t in range(1, NTAB):
                    acc = acc + bufs[par][t][r, sl]
                outb[par][r, sl] = (acc + cc[v]) * (1.0 / 7.0)
            return cc

        lax.fori_loop(0, C, body, tuple(const))
        o_cp[ch] = pltpu.async_copy(
            outb[par], out_hbm.at[pl.ds((chunk0 + ch) * C, C)], osem[par])
    for cp in o_cp.values():
        cp.wait()


@functools.cache
def _sc_gather_sum():
    return pl.kernel(
        _sc_body,
        out_type=jax.ShapeDtypeStruct((NPOS, EMBED_DIM), jnp.float32),
        mesh=plsc.VectorSubcoreMesh(core_axis_name="c", subcore_axis_name="s"),
        scratch_types=(
            [pltpu.VMEM((NTAB, C), jnp.int32) for _ in range(2)]
            + [pltpu.VMEM((C, EMBED_DIM), jnp.float32)
               for _ in range(2 * NTAB)]
            + [pltpu.VMEM((C, EMBED_DIM), jnp.float32) for _ in range(2)]
            + [pltpu.VMEM((MAX_N - 2, EMBED_DIM), jnp.float32)]
            + [pltpu.SemaphoreType.DMA for _ in range(4)]
        ),
    )


def kernel(tokens, main_w, shared_w, size_w):
    tokens = tokens.astype(jnp.int32)
    idx = _compute_indices(tokens)
    idx = idx.reshape(NTAB, NCHUNKS, C).transpose(1, 0, 2)
    out = _sc_gather_sum()(idx, main_w, shared_w, size_w)
    return out.reshape(BSZ, SEQ, EMBED_DIM)


# speedup vs baseline: 1.9967x; 1.0433x over previous
import functools

import jax
import jax.numpy as jnp
from jax import lax
from jax.experimental import pallas as pl
from jax.experimental.pallas import tpu as pltpu
from jax.experimental.pallas import tpu_sc as plsc

EMBED_DIM = 128
MAX_N = 8
NUM_BUCKETS = 500000
HASH_BASE = 260
HASH_MOD = 1 << 23
MASK = HASH_MOD - 1

BSZ = 16
SEQ = 2048
NPOS = BSZ * SEQ
NTAB = 1 + (MAX_N - 2)
NW = 32
C = 32
NCHUNKS = NPOS // C
CH_PER_W = NCHUNKS // NW


def _idx_body(tok_ref, idx_ref):
    t = tok_ref[...]
    idx_ref[0] = t
    col = lax.broadcasted_iota(jnp.int32, (BSZ, SEQ), 1)
    s_acc = t
    for n in range(2, MAX_N + 1):
        d = n - 1
        p = pow(HASH_BASE, d, HASH_MOD)
        t_shift = jnp.concatenate(
            [jnp.zeros((BSZ, d), jnp.int32), t[:, : SEQ - d]], axis=1)
        s_acc = s_acc + ((t_shift * p) & MASK)
        if n >= 3:
            idx = (s_acc & MASK) % NUM_BUCKETS
            idx_ref[n - 2] = jnp.where(col < n - 1, 0, idx)


def _compute_indices(tokens):
    return pl.pallas_call(
        _idx_body,
        out_shape=jax.ShapeDtypeStruct((NTAB, BSZ, SEQ), jnp.int32),
    )(tokens)


DEPTH = 3


def _sc_body(idx_hbm, main_hbm, shared_hbm, size_hbm, out_hbm, *s):
    idx_v = s[0]
    bufs = tuple(s[1 + d * NTAB:1 + (d + 1) * NTAB] for d in range(DEPTH))
    outb = (s[1 + DEPTH * NTAB], s[2 + DEPTH * NTAB])
    size_v = s[3 + DEPTH * NTAB]
    gsem = s[4 + DEPTH * NTAB]
    osem = (s[5 + DEPTH * NTAB], s[6 + DEPTH * NTAB])
    wid = lax.axis_index("s") * 2 + lax.axis_index("c")

    pltpu.sync_copy(idx_hbm.at[wid], idx_v)

    pltpu.sync_copy(size_hbm, size_v)
    const = []
    for v in range(EMBED_DIM // 16):
        sl = pl.ds(v * 16, 16)
        cv = size_v[0, sl]
        for t in range(1, MAX_N - 2):
            cv = cv + size_v[t, sl]
        const.append(cv)

    def fire_gathers(ch):
        cps = []
        for t in range(NTAB):
            table = main_hbm if t == 0 else shared_hbm
            cps.append(pltpu.async_copy(
                table.at[idx_v.at[ch, t]], bufs[ch % DEPTH][t], gsem))
        return cps

    chunk0 = wid * CH_PER_W
    g = {ch: fire_gathers(ch) for ch in range(min(DEPTH, CH_PER_W))}
    o_cp = {}

    for ch in range(CH_PER_W):
        par = ch % 2
        for cp in g.pop(ch):
            cp.wait()
        if ch - 2 in o_cp:
            o_cp.pop(ch - 2).wait()

        def body(r, cc, ch=ch, par=par):
            for v in range(EMBED_DIM // 16):
                sl = pl.ds(v * 16, 16)
                acc = bufs[ch % DEPTH][0][r, sl]
                for t in range(1, NTAB):
                    acc = acc + bufs[ch % DEPTH][t][r, sl]
                outb[par][r, sl] = (acc + cc[v]) * (1.0 / 7.0)
            return cc

        lax.fori_loop(0, C, body, tuple(const))
        o_cp[ch] = pltpu.async_copy(
            outb[par], out_hbm.at[pl.ds((chunk0 + ch) * C, C)], osem[par])
        if ch + DEPTH < CH_PER_W:
            g[ch + DEPTH] = fire_gathers(ch + DEPTH)
    for cp in o_cp.values():
        cp.wait()


@functools.cache
def _sc_gather_sum():
    return pl.kernel(
        _sc_body,
        out_type=jax.ShapeDtypeStruct((NPOS, EMBED_DIM), jnp.float32),
        mesh=plsc.VectorSubcoreMesh(core_axis_name="c", subcore_axis_name="s"),
        scratch_types=(
            [pltpu.VMEM((CH_PER_W, NTAB, C), jnp.int32)]
            + [pltpu.VMEM((C, EMBED_DIM), jnp.float32)
               for _ in range(DEPTH * NTAB)]
            + [pltpu.VMEM((C, EMBED_DIM), jnp.float32) for _ in range(2)]
            + [pltpu.VMEM((MAX_N - 2, EMBED_DIM), jnp.float32)]
            + [pltpu.SemaphoreType.DMA for _ in range(3)]
        ),
    )


def kernel(tokens, main_w, shared_w, size_w):
    tokens = tokens.astype(jnp.int32)
    idx = _compute_indices(tokens)
    idx = (idx.reshape(NTAB, NCHUNKS, C).transpose(1, 0, 2)
           .reshape(NW, CH_PER_W, NTAB, C))
    out = _sc_gather_sum()(idx, main_w, shared_w, size_w)
    return out.reshape(BSZ, SEQ, EMBED_DIM)


# speedup vs baseline: 2.0880x; 1.0457x over previous
import functools

import jax
import jax.numpy as jnp
from jax import lax
from jax.experimental import pallas as pl
from jax.experimental.pallas import tpu as pltpu
from jax.experimental.pallas import tpu_sc as plsc

EMBED_DIM = 128
MAX_N = 8
NUM_BUCKETS = 500000
HASH_BASE = 260
HASH_MOD = 1 << 23
MASK = HASH_MOD - 1

BSZ = 16
SEQ = 2048
NPOS = BSZ * SEQ
NTAB = 1 + (MAX_N - 2)
NW = 32
C = 32
NCHUNKS = NPOS // C
CH_PER_W = NCHUNKS // NW
P_PER_W = NPOS // NW


def _idx_body(tok_ref, idx_ref):
    t = tok_ref[...]
    idx_ref[0] = t
    col = lax.broadcasted_iota(jnp.int32, (BSZ, SEQ), 1)
    s_acc = t
    for n in range(2, MAX_N + 1):
        d = n - 1
        p = pow(HASH_BASE, d, HASH_MOD)
        t_shift = jnp.concatenate(
            [jnp.zeros((BSZ, d), jnp.int32), t[:, : SEQ - d]], axis=1)
        s_acc = s_acc + ((t_shift * p) & MASK)
        if n >= 3:
            idx = (s_acc & MASK) % NUM_BUCKETS
            idx_ref[n - 2] = jnp.where(col < n - 1, 0, idx)


def _compute_indices(tokens):
    return pl.pallas_call(
        _idx_body,
        out_shape=jax.ShapeDtypeStruct((NTAB, BSZ, SEQ), jnp.int32),
    )(tokens)


DEPTH = 3


def _sc_body(idx_hbm, main_hbm, shared_hbm, size_hbm, out_hbm, *s):
    idx_v = s[0]
    bufs = tuple(s[1 + d * NTAB:1 + (d + 1) * NTAB] for d in range(DEPTH))
    outb = (s[1 + DEPTH * NTAB], s[2 + DEPTH * NTAB])
    size_v = s[3 + DEPTH * NTAB]
    gsem = s[4 + DEPTH * NTAB]
    osem = (s[5 + DEPTH * NTAB], s[6 + DEPTH * NTAB])
    wid = lax.axis_index("c") * 16 + lax.axis_index("s")

    pltpu.sync_copy(idx_hbm.at[wid], idx_v)

    pltpu.sync_copy(size_hbm, size_v)
    const = []
    for v in range(EMBED_DIM // 16):
        sl = pl.ds(v * 16, 16)
        cv = size_v[0, sl]
        for t in range(1, MAX_N - 2):
            cv = cv + size_v[t, sl]
        const.append(cv)

    def fire_gathers(ch):
        cps = []
        for t in range(NTAB):
            table = main_hbm if t == 0 else shared_hbm
            cps.append(pltpu.async_copy(
                table.at[idx_v.at[ch, t]], bufs[ch % DEPTH][t], gsem))
        return cps

    chunk0 = wid * CH_PER_W
    g = {ch: fire_gathers(ch) for ch in range(min(DEPTH, CH_PER_W))}
    o_cp = {}

    for ch in range(CH_PER_W):
        par = ch % 2
        for cp in g.pop(ch):
            cp.wait()
        if ch - 2 in o_cp:
            o_cp.pop(ch - 2).wait()

        def body(r, cc, ch=ch, par=par):
            for v in range(EMBED_DIM // 16):
                sl = pl.ds(v * 16, 16)
                acc = bufs[ch % DEPTH][0][r, sl]
                for t in range(1, NTAB):
                    acc = acc + bufs[ch % DEPTH][t][r, sl]
                outb[par][r, sl] = (acc + cc[v]) * (1.0 / 7.0)
            return cc

        lax.fori_loop(0, C, body, tuple(const))
        o_cp[ch] = pltpu.async_copy(
            outb[par], out_hbm.at[pl.ds((chunk0 + ch) * C, C)], osem[par])
        if ch + DEPTH < CH_PER_W:
            g[ch + DEPTH] = fire_gathers(ch + DEPTH)
    for cp in o_cp.values():
        cp.wait()


@functools.cache
def _sc_gather_sum():
    return pl.kernel(
        _sc_body,
        out_type=jax.ShapeDtypeStruct((NPOS, EMBED_DIM), jnp.float32),
        mesh=plsc.VectorSubcoreMesh(core_axis_name="c", subcore_axis_name="s"),
        scratch_types=(
            [pltpu.VMEM((CH_PER_W, NTAB, C), jnp.int32)]
            + [pltpu.VMEM((C, EMBED_DIM), jnp.float32)
               for _ in range(DEPTH * NTAB)]
            + [pltpu.VMEM((C, EMBED_DIM), jnp.float32) for _ in range(2)]
            + [pltpu.VMEM((MAX_N - 2, EMBED_DIM), jnp.float32)]
            + [pltpu.SemaphoreType.DMA for _ in range(3)]
        ),
    )


def kernel(tokens, main_w, shared_w, size_w):
    tokens = tokens.astype(jnp.int32)
    idx = _compute_indices(tokens)
    idx = (idx.reshape(NTAB, NCHUNKS, C).transpose(1, 0, 2)
           .reshape(NW, CH_PER_W, NTAB, C))
    out = _sc_gather_sum()(idx, main_w, shared_w, size_w)
    return out.reshape(BSZ, SEQ, EMBED_DIM)


# speedup vs baseline: 2.3265x; 1.1142x over previous
import functools

import jax
import jax.numpy as jnp
from jax import lax
from jax.experimental import pallas as pl
from jax.experimental.pallas import tpu as pltpu
from jax.experimental.pallas import tpu_sc as plsc

EMBED_DIM = 128
MAX_N = 8
NUM_BUCKETS = 500000
HASH_BASE = 260
HASH_MOD = 1 << 23
MASK = HASH_MOD - 1

BSZ = 16
SEQ = 2048
NPOS = BSZ * SEQ
NTAB = 1 + (MAX_N - 2)
NW = 32
C = 32
CH_PER_W = NPOS // (NW * C)
P_PER_W = NPOS // NW
NBUF = 3
NVEC = EMBED_DIM // 16
TLEN = P_PER_W + 16
TLEN2 = TLEN + 16
NG = TLEN // 16


def _pad_body(t_ref, o_ref):
    z = jnp.zeros((1, 128), jnp.int32)
    o_ref[pl.ds(0, 1), :] = z
    o_ref[pl.ds(1, 256), :] = t_ref[...]
    o_ref[pl.ds(257, 1), :] = z


def _pad_tokens(tokens2d):
    return pl.pallas_call(
        _pad_body,
        out_shape=jax.ShapeDtypeStruct((258, 128), jnp.int32),
    )(tokens2d)


def _mod_buckets(x):
    for k in (16, 8, 4, 2, 1):
        kd = k * NUM_BUCKETS
        x = jnp.where(x >= kd, x - kd, x)
    return x


def _sc_body(tok_hbm, main_hbm, shared_hbm, size_hbm, out_hbm, *s):
    idx_v = s[0]
    bufs = tuple(s[1 + b * NTAB:1 + (b + 1) * NTAB] for b in range(NBUF))
    outb = tuple(s[1 + NBUF * NTAB + b] for b in range(NBUF))
    size_v = s[1 + NBUF * NTAB + NBUF]
    tok_v = s[2 + NBUF * NTAB + NBUF]
    hsc = s[3 + NBUF * NTAB + NBUF]
    gsem = s[4 + NBUF * NTAB + NBUF:4 + NBUF * NTAB + 2 * NBUF]
    osem = s[4 + NBUF * NTAB + 2 * NBUF:4 + NBUF * NTAB + 3 * NBUF]

    sid = lax.axis_index("s")
    wid = lax.axis_index("c") * 16 + sid
    base = wid * P_PER_W
    srow = lax.rem(wid, 2) * P_PER_W

    pltpu.sync_copy(tok_hbm.at[pl.ds(base + 112, TLEN)], tok_v)
    pltpu.sync_copy(size_hbm, size_v)

    const = []
    for v in range(NVEC):
        sl = pl.ds(v * 16, 16)
        cv = size_v[0, sl]
        for t in range(1, MAX_N - 2):
            cv = cv + size_v[t, sl]
        const.append(cv)

    def h1_body(g, c):
        hsc[pl.ds(16 + g * 16, 16)] = tok_v[pl.ds(g * 16, 16)]
        return c
    lax.fori_loop(0, NG, h1_body, 0)

    lanes = lax.broadcasted_iota(jnp.int32, (16,), 0)

    def hash_group(g, n, store_idx, b_slot=None, gg=None):
        off = g * 16
        hp = hsc[pl.ds((n - 2) * TLEN2 + 15 + off, 16)]
        tv = tok_v[pl.ds(off, 16)]
        hc = (hp * HASH_BASE + tv) & MASK
        hsc[pl.ds((n - 1) * TLEN2 + 16 + off, 16)] = hc
        if store_idx:
            @pl.when(n >= 3)
            def _():
                x = _mod_buckets(hc)
                s_vec = srow + (off - 16) + lanes
                idx_v[b_slot, n - 2, pl.ds(gg * 16, 16)] = jnp.where(
                    s_vec < n - 1, 0, x)

    def hash_halo():
        def body(n, c):
            hash_group(jnp.int32(0), n, False)
            return c
        lax.fori_loop(2, MAX_N + 1, body, 0)

    def hash_chunk(chd, b):
        for gg in range(2):
            g = 2 * chd + 1 + gg
            sl16 = pl.ds(g * 16, 16)
            idx_v[b, 0, pl.ds(gg * 16, 16)] = tok_v[sl16]

        def body(n, c):
            for gg in range(2):
                hash_group(2 * chd + 1 + gg, n, True, b, gg)
            return c
        lax.fori_loop(2, MAX_N + 1, body, 0)

    def fire_gathers(b):
        for t in range(NTAB):
            table = main_hbm if t == 0 else shared_hbm
            pltpu.async_copy(table.at[idx_v.at[b, t]], bufs[b][t], gsem[b])

    def drain_gathers(b):
        for t in range(NTAB):
            pltpu.make_async_copy(
                shared_hbm.at[pl.ds(0, C)], bufs[b][t], gsem[b]).wait()

    def drain_out(b):
        pltpu.make_async_copy(
            outb[b], out_hbm.at[pl.ds(0, C)], osem[b]).wait()

    hash_halo()
    for ch in range(NBUF):
        hash_chunk(jnp.int32(ch), ch)
        fire_gathers(ch)

    def outer(gi, c):
        for b in range(NBUF):
            ch = gi * NBUF + b

            @pl.when(ch < CH_PER_W)
            def _():
                drain_gathers(b)

                @pl.when(ch >= NBUF)
                def _():
                    drain_out(b)

                def body(r, cc):
                    for v in range(NVEC):
                        sl = pl.ds(v * 16, 16)
                        acc = bufs[b][0][r, sl]
                        for t in range(1, NTAB):
                            acc = acc + bufs[b][t][r, sl]
                        outb[b][r, sl] = (acc + cc[v]) * (1.0 / 7.0)
                    return cc
                lax.fori_loop(0, C, body, tuple(const))
                pltpu.async_copy(
                    outb[b], out_hbm.at[pl.ds(base + ch * C, C)], osem[b])

                @pl.when(ch + NBUF < CH_PER_W)
                def _():
                    hash_chunk(ch + NBUF, b)
                    fire_gathers(b)
        return c

    nit = (CH_PER_W + NBUF - 1) // NBUF
    lax.fori_loop(0, nit, outer, 0)
    for b in range(NBUF):
        drain_out(b)


@functools.cache
def _sc_embed():
    return pl.kernel(
        _sc_body,
        out_type=jax.ShapeDtypeStruct((NPOS, EMBED_DIM), jnp.float32),
        mesh=plsc.VectorSubcoreMesh(core_axis_name="c", subcore_axis_name="s"),
        scratch_types=(
            [pltpu.VMEM((NBUF, NTAB, C), jnp.int32)]
            + [pltpu.VMEM((C, EMBED_DIM), jnp.float32)
               for _ in range(NBUF * NTAB)]
            + [pltpu.VMEM((C, EMBED_DIM), jnp.float32) for _ in range(NBUF)]
            + [pltpu.VMEM((MAX_N - 2, EMBED_DIM), jnp.float32),
               pltpu.VMEM((TLEN,), jnp.int32),
               pltpu.VMEM((MAX_N * TLEN2,), jnp.int32)]
            + [pltpu.SemaphoreType.DMA for _ in range(2 * NBUF)]
        ),
    )


def kernel(tokens, main_w, shared_w, size_w):
    tokens = tokens.astype(jnp.int32).reshape(256, 128)
    tokp = _pad_tokens(tokens).reshape(258 * 128)
    out = _sc_embed()(tokp, main_w, shared_w, size_w)
    return out.reshape(BSZ, SEQ, EMBED_DIM)
